# Initial kernel scaffold; baseline (speedup 1.0000x reference)
import jax
import jax.numpy as jnp
from jax.experimental import pallas as pl


def kernel(x, conv1_w, conv1_s, conv2_b0_conv1_w, conv2_b0_conv1_s, conv2_b0_conv2_w, conv2_b0_conv2_s, conv2_b1_down_w, conv2_b1_down_s, conv2_b1_conv1_w, conv2_b1_conv1_s, conv2_b1_conv2_w, conv2_b1_conv2_s, conv2_b1_conv3_w, conv2_b1_conv3_s, conv2_b2_conv1_w, conv2_b2_conv1_s, conv2_b2_conv2_w, conv2_b2_conv2_s, conv2_b2_conv3_w, conv2_b2_conv3_s, conv3_b0_down_w, conv3_b0_down_s, conv3_b0_conv1_w, conv3_b0_conv1_s, conv3_b0_conv2_w, conv3_b0_conv2_s, conv3_b1_down_w, conv3_b1_down_s, conv3_b1_conv1_w, conv3_b1_conv1_s, conv3_b1_conv2_w, conv3_b1_conv2_s, conv3_b1_conv3_w, conv3_b1_conv3_s, conv3_b2_conv1_w, conv3_b2_conv1_s, conv3_b2_conv2_w, conv3_b2_conv2_s, conv3_b2_conv3_w, conv3_b2_conv3_s, conv3_b3_conv1_w, conv3_b3_conv1_s, conv3_b3_conv2_w, conv3_b3_conv2_s, conv3_b3_conv3_w, conv3_b3_conv3_s, conv4_b0_down_w, conv4_b0_down_s, conv4_b0_conv1_w, conv4_b0_conv1_s, conv4_b0_conv2_w, conv4_b0_conv2_s, conv4_b1_down_w, conv4_b1_down_s, conv4_b1_conv1_w, conv4_b1_conv1_s, conv4_b1_conv2_w, conv4_b1_conv2_s, conv4_b1_conv3_w, conv4_b1_conv3_s, conv4_b2_conv1_w, conv4_b2_conv1_s, conv4_b2_conv2_w, conv4_b2_conv2_s, conv4_b2_conv3_w, conv4_b2_conv3_s, conv4_b3_conv1_w, conv4_b3_conv1_s, conv4_b3_conv2_w, conv4_b3_conv2_s, conv4_b3_conv3_w, conv4_b3_conv3_s, conv4_b4_conv1_w, conv4_b4_conv1_s, conv4_b4_conv2_w, conv4_b4_conv2_s, conv4_b4_conv3_w, conv4_b4_conv3_s, conv4_b5_conv1_w, conv4_b5_conv1_s, conv4_b5_conv2_w, conv4_b5_conv2_s, conv4_b5_conv3_w, conv4_b5_conv3_s, conv5_b0_down_w, conv5_b0_down_s, conv5_b0_conv1_w, conv5_b0_conv1_s, conv5_b0_conv2_w, conv5_b0_conv2_s, conv5_b1_down_w, conv5_b1_down_s, conv5_b1_conv1_w, conv5_b1_conv1_s, conv5_b1_conv2_w, conv5_b1_conv2_s, conv5_b1_conv3_w, conv5_b1_conv3_s, conv5_b2_conv1_w, conv5_b2_conv1_s, conv5_b2_conv2_w, conv5_b2_conv2_s, conv5_b2_conv3_w, conv5_b2_conv3_s, fc_w, fc_s, fc1_w, fc1_s, fc2_w, fc2_s):
    raise NotImplementedError("write your pallas kernel here")


# speedup vs baseline: 4.7057x; 4.7057x over previous
import functools

import jax
import jax.numpy as jnp
from jax.experimental import pallas as pl
from jax.experimental.pallas import tpu as pltpu

_BF = jnp.bfloat16
_TAPS = tuple((i, j) for i in range(3) for j in range(3))


def _ref_tk(k):
    kp = (k + 127) // 128 * 128
    if kp <= 1024:
        return kp
    for t in (512, 384, 256, 128):
        if kp % t == 0:
            return t
    return 512


def _chunked_dot(a, w_ref, kk, cout):
    return jnp.dot(a[:, :kk], w_ref[:kk, :cout],
                   preferred_element_type=jnp.float32)


def _segment_body(*refs, blocks, bn, nw):
    x_ref = refs[0]
    w_refs = refs[1:1 + nw]
    o_ref = refs[1 + nw]
    xpad_ref = refs[2 + nw]
    patch_ref = refs[3 + nw]

    xpad_ref[...] = jnp.zeros_like(xpad_ref)
    cursor = [0]

    def conv(a4, ks, cin, cout, res=None, relu=True):
        i = cursor[0]
        cursor[0] = i + 2
        w_ref, s_ref = w_refs[i], w_refs[i + 1]
        if ks == 3:
            xpad_ref[:, 1:9, 1:9, :cin] = a4
            for t, (di, dj) in enumerate(_TAPS):
                patch_ref[:, :, :, t * cin:(t + 1) * cin] = (
                    xpad_ref[:, di:di + 8, dj:dj + 8, :cin])
            a = patch_ref[:, :, :, :9 * cin].reshape(bn * 64, 9 * cin)
            kk = 9 * cin
        else:
            a = a4.reshape(bn * 64, cin)
            kk = cin
        y = _chunked_dot(a, w_ref, kk, cout)
        y = y + s_ref[0:1, :cout]
        if res is not None:
            y = y + res.reshape(bn * 64, cout).astype(jnp.float32)
        if relu:
            y = jnp.maximum(y, 0.0)
        return y.astype(_BF).reshape(bn, 8, 8, cout)

    cur = x_ref[...]
    for kind, convs in blocks:
        cd = dict((c[0], c[1:]) for c in convs)
        sc = conv(cur, *cd['down'], relu=False) if 'down' in cd else cur
        h = conv(cur, *cd['conv1'])
        if kind == 'basic':
            cur = conv(h, *cd['conv2'], res=sc)
        else:
            h = conv(h, *cd['conv2'])
            cur = conv(h, *cd['conv3'], res=sc)
    o_ref[...] = cur


def _run_segment(x, blocks, weights, bn):
    n, _, _, cin0 = x.shape
    bn = min(bn, n)
    cout = blocks[-1][1][-1][3]
    cmax = max(c[2] for _, convs in blocks for c in convs if c[1] == 3)
    nw = len(weights)
    in_specs = [pl.BlockSpec((bn, 8, 8, cin0), lambda b: (b, 0, 0, 0))]
    for wgt in weights:
        in_specs.append(pl.BlockSpec(wgt.shape, lambda b: (0, 0)))
    body = functools.partial(_segment_body, blocks=blocks, bn=bn, nw=nw)
    return pl.pallas_call(
        body,
        out_shape=jax.ShapeDtypeStruct((n, 8, 8, cout), _BF),
        grid=(n // bn,),
        in_specs=in_specs,
        out_specs=pl.BlockSpec((bn, 8, 8, cout), lambda b: (b, 0, 0, 0)),
        scratch_shapes=[pltpu.VMEM((bn, 10, 10, cmax), _BF),
                        pltpu.VMEM((bn, 8, 8, 9 * cmax), _BF)],
        compiler_params=pltpu.CompilerParams(
            dimension_semantics=("parallel",)),
    )(x, *weights)


def _conv1_pool_body(a_ref, w_ref, s_ref, o_ref, mp_ref, *, bn):
    y = jnp.dot(a_ref[...], w_ref[:, :64], preferred_element_type=jnp.float32)
    y = jnp.maximum(y + s_ref[0:1, :64], 0.0).astype(_BF)
    mp_ref[...] = jnp.zeros_like(mp_ref)
    mp_ref[:, 1:17, 1:17, :] = y.reshape(bn, 16, 16, 64).astype(jnp.float32)
    m = mp_ref[:, 0:16:2, 0:16:2, :]
    for i, j in _TAPS:
        if (i, j) != (0, 0):
            m = jnp.maximum(m, mp_ref[:, i:i + 16:2, j:j + 16:2, :])
    o_ref[...] = m.astype(_BF)


def _conv1_pool(a, w, s, n, bn):
    bn = min(bn, n)
    return pl.pallas_call(
        functools.partial(_conv1_pool_body, bn=bn),
        out_shape=jax.ShapeDtypeStruct((n, 8, 8, 64), _BF),
        grid=(n // bn,),
        in_specs=[pl.BlockSpec((bn * 256, 256), lambda b: (b, 0)),
                  pl.BlockSpec(w.shape, lambda b: (0, 0)),
                  pl.BlockSpec(s.shape, lambda b: (0, 0))],
        out_specs=pl.BlockSpec((bn, 8, 8, 64), lambda b: (b, 0, 0, 0)),
        scratch_shapes=[pltpu.VMEM((bn, 18, 18, 64), jnp.float32)],
        compiler_params=pltpu.CompilerParams(
            dimension_semantics=("parallel",)),
    )(a, w, s)


def _fc_body(a_ref, w_ref, s_ref, o_ref, acc_ref):
    k = pl.program_id(1)

    @pl.when(k == 0)
    def _():
        acc_ref[...] = jnp.zeros_like(acc_ref)

    acc_ref[...] += jnp.dot(a_ref[...], w_ref[...],
                            preferred_element_type=jnp.float32)

    @pl.when(k == pl.num_programs(1) - 1)
    def _():
        o_ref[...] = jnp.maximum(acc_ref[...] + s_ref[...], 0.0).astype(_BF)


def _fc_stream(a, w, s, tn=1024, tk=4096):
    m, kdim = a.shape
    ncols = w.shape[1]
    return pl.pallas_call(
        _fc_body,
        out_shape=jax.ShapeDtypeStruct((m, ncols), _BF),
        grid=(ncols // tn, kdim // tk),
        in_specs=[pl.BlockSpec((m, tk), lambda nb, kb: (0, kb)),
                  pl.BlockSpec((tk, tn), lambda nb, kb: (kb, nb)),
                  pl.BlockSpec((1, tn), lambda nb, kb: (0, nb))],
        out_specs=pl.BlockSpec((m, tn), lambda nb, kb: (0, nb)),
        scratch_shapes=[pltpu.VMEM((m, tn), jnp.float32)],
        compiler_params=pltpu.CompilerParams(
            dimension_semantics=("parallel", "arbitrary")),
    )(a, w, s)


def _head_body(h_ref, w1_ref, s1_ref, w2_ref, s2_ref, o_ref):
    y1 = jnp.dot(h_ref[...], w1_ref[...], preferred_element_type=jnp.float32)
    y1 = jnp.maximum(y1 + s1_ref[...], 0.0).astype(_BF)
    o_ref[...] = (jnp.dot(y1, w2_ref[...], preferred_element_type=jnp.float32)
                  + s2_ref[...])


def _head(h, w1, s1, w2, s2):
    return pl.pallas_call(
        _head_body,
        out_shape=jax.ShapeDtypeStruct((h.shape[0], w2.shape[1]),
                                       jnp.float32),
    )(h, w1, s1, w2, s2)


def _im2col7(xh):
    n = xh.shape[0]
    xp = jnp.pad(xh, ((0, 0), (3, 3), (3, 3), (0, 0)))
    cols = [xp[:, i:i + 32:2, j:j + 32:2, :]
            for i in range(7) for j in range(7)]
    p = jnp.stack(cols, axis=3).reshape(n * 256, 147)
    return jnp.pad(p, ((0, 0), (0, 109)))


def kernel(x, conv1_w, conv1_s, conv2_b0_conv1_w, conv2_b0_conv1_s, conv2_b0_conv2_w, conv2_b0_conv2_s, conv2_b1_down_w, conv2_b1_down_s, conv2_b1_conv1_w, conv2_b1_conv1_s, conv2_b1_conv2_w, conv2_b1_conv2_s, conv2_b1_conv3_w, conv2_b1_conv3_s, conv2_b2_conv1_w, conv2_b2_conv1_s, conv2_b2_conv2_w, conv2_b2_conv2_s, conv2_b2_conv3_w, conv2_b2_conv3_s, conv3_b0_down_w, conv3_b0_down_s, conv3_b0_conv1_w, conv3_b0_conv1_s, conv3_b0_conv2_w, conv3_b0_conv2_s, conv3_b1_down_w, conv3_b1_down_s, conv3_b1_conv1_w, conv3_b1_conv1_s, conv3_b1_conv2_w, conv3_b1_conv2_s, conv3_b1_conv3_w, conv3_b1_conv3_s, conv3_b2_conv1_w, conv3_b2_conv1_s, conv3_b2_conv2_w, conv3_b2_conv2_s, conv3_b2_conv3_w, conv3_b2_conv3_s, conv3_b3_conv1_w, conv3_b3_conv1_s, conv3_b3_conv2_w, conv3_b3_conv2_s, conv3_b3_conv3_w, conv3_b3_conv3_s, conv4_b0_down_w, conv4_b0_down_s, conv4_b0_conv1_w, conv4_b0_conv1_s, conv4_b0_conv2_w, conv4_b0_conv2_s, conv4_b1_down_w, conv4_b1_down_s, conv4_b1_conv1_w, conv4_b1_conv1_s, conv4_b1_conv2_w, conv4_b1_conv2_s, conv4_b1_conv3_w, conv4_b1_conv3_s, conv4_b2_conv1_w, conv4_b2_conv1_s, conv4_b2_conv2_w, conv4_b2_conv2_s, conv4_b2_conv3_w, conv4_b2_conv3_s, conv4_b3_conv1_w, conv4_b3_conv1_s, conv4_b3_conv2_w, conv4_b3_conv2_s, conv4_b3_conv3_w, conv4_b3_conv3_s, conv4_b4_conv1_w, conv4_b4_conv1_s, conv4_b4_conv2_w, conv4_b4_conv2_s, conv4_b4_conv3_w, conv4_b4_conv3_s, conv4_b5_conv1_w, conv4_b5_conv1_s, conv4_b5_conv2_w, conv4_b5_conv2_s, conv4_b5_conv3_w, conv4_b5_conv3_s, conv5_b0_down_w, conv5_b0_down_s, conv5_b0_conv1_w, conv5_b0_conv1_s, conv5_b0_conv2_w, conv5_b0_conv2_s, conv5_b1_down_w, conv5_b1_down_s, conv5_b1_conv1_w, conv5_b1_conv1_s, conv5_b1_conv2_w, conv5_b1_conv2_s, conv5_b1_conv3_w, conv5_b1_conv3_s, conv5_b2_conv1_w, conv5_b2_conv1_s, conv5_b2_conv2_w, conv5_b2_conv2_s, conv5_b2_conv3_w, conv5_b2_conv3_s, fc_w, fc_s, fc1_w, fc1_s, fc2_w, fc2_s):
    n = x.shape[0]
    xh = jnp.transpose(x, (0, 2, 3, 1)).astype(_BF)
    a1 = _im2col7(xh)
    h = _conv1_pool(a1, conv1_w, conv1_s, n, 16)

    segs = [
        ((('basic', (('conv1', 3, 64, 64), ('conv2', 3, 64, 64))),),
         [conv2_b0_conv1_w, conv2_b0_conv1_s,
          conv2_b0_conv2_w, conv2_b0_conv2_s], 16),
        ((('bneck', (('down', 1, 64, 256), ('conv1', 1, 64, 64),
                     ('conv2', 3, 64, 64), ('conv3', 1, 64, 256))),),
         [conv2_b1_down_w, conv2_b1_down_s, conv2_b1_conv1_w, conv2_b1_conv1_s,
          conv2_b1_conv2_w, conv2_b1_conv2_s,
          conv2_b1_conv3_w, conv2_b1_conv3_s], 16),
        ((('bneck', (('conv1', 1, 256, 64), ('conv2', 3, 64, 64),
                     ('conv3', 1, 64, 256))),),
         [conv2_b2_conv1_w, conv2_b2_conv1_s, conv2_b2_conv2_w,
          conv2_b2_conv2_s, conv2_b2_conv3_w, conv2_b2_conv3_s], 16),
        ((('basic', (('down', 1, 256, 128), ('conv1', 3, 256, 128),
                     ('conv2', 3, 128, 128))),),
         [conv3_b0_down_w, conv3_b0_down_s, conv3_b0_conv1_w, conv3_b0_conv1_s,
          conv3_b0_conv2_w, conv3_b0_conv2_s], 16),
        ((('bneck', (('down', 1, 128, 512), ('conv1', 1, 128, 128),
                     ('conv2', 3, 128, 128), ('conv3', 1, 128, 512))),),
         [conv3_b1_down_w, conv3_b1_down_s, conv3_b1_conv1_w, conv3_b1_conv1_s,
          conv3_b1_conv2_w, conv3_b1_conv2_s,
          conv3_b1_conv3_w, conv3_b1_conv3_s], 16),
        ((('bneck', (('conv1', 1, 512, 128), ('conv2', 3, 128, 128),
                     ('conv3', 1, 128, 512))),),
         [conv3_b2_conv1_w, conv3_b2_conv1_s, conv3_b2_conv2_w,
          conv3_b2_conv2_s, conv3_b2_conv3_w, conv3_b2_conv3_s], 16),
        ((('bneck', (('conv1', 1, 512, 128), ('conv2', 3, 128, 128),
                     ('conv3', 1, 128, 512))),),
         [conv3_b3_conv1_w, conv3_b3_conv1_s, conv3_b3_conv2_w,
          conv3_b3_conv2_s, conv3_b3_conv3_w, conv3_b3_conv3_s], 16),
        ((('basic', (('down', 1, 512, 256), ('conv1', 3, 512, 256),
                     ('conv2', 3, 256, 256))),),
         [conv4_b0_down_w, conv4_b0_down_s, conv4_b0_conv1_w, conv4_b0_conv1_s,
          conv4_b0_conv2_w, conv4_b0_conv2_s], 16),
        ((('bneck', (('down', 1, 256, 1024), ('conv1', 1, 256, 256),
                     ('conv2', 3, 256, 256), ('conv3', 1, 256, 1024))),),
         [conv4_b1_down_w, conv4_b1_down_s, conv4_b1_conv1_w, conv4_b1_conv1_s,
          conv4_b1_conv2_w, conv4_b1_conv2_s,
          conv4_b1_conv3_w, conv4_b1_conv3_s], 16),
        ((('bneck', (('conv1', 1, 1024, 256), ('conv2', 3, 256, 256),
                     ('conv3', 1, 256, 1024))),),
         [conv4_b2_conv1_w, conv4_b2_conv1_s, conv4_b2_conv2_w,
          conv4_b2_conv2_s, conv4_b2_conv3_w, conv4_b2_conv3_s], 16),
        ((('bneck', (('conv1', 1, 1024, 256), ('conv2', 3, 256, 256),
                     ('conv3', 1, 256, 1024))),),
         [conv4_b3_conv1_w, conv4_b3_conv1_s, conv4_b3_conv2_w,
          conv4_b3_conv2_s, conv4_b3_conv3_w, conv4_b3_conv3_s], 16),
        ((('bneck', (('conv1', 1, 1024, 256), ('conv2', 3, 256, 256),
                     ('conv3', 1, 256, 1024))),),
         [conv4_b4_conv1_w, conv4_b4_conv1_s, conv4_b4_conv2_w,
          conv4_b4_conv2_s, conv4_b4_conv3_w, conv4_b4_conv3_s], 16),
        ((('bneck', (('conv1', 1, 1024, 256), ('conv2', 3, 256, 256),
                     ('conv3', 1, 256, 1024))),),
         [conv4_b5_conv1_w, conv4_b5_conv1_s, conv4_b5_conv2_w,
          conv4_b5_conv2_s, conv4_b5_conv3_w, conv4_b5_conv3_s], 16),
        ((('basic', (('down', 1, 1024, 512), ('conv1', 3, 1024, 512),
                     ('conv2', 3, 512, 512))),),
         [conv5_b0_down_w, conv5_b0_down_s, conv5_b0_conv1_w, conv5_b0_conv1_s,
          conv5_b0_conv2_w, conv5_b0_conv2_s], 8),
        ((('bneck', (('down', 1, 512, 2048), ('conv1', 1, 512, 512),
                     ('conv2', 3, 512, 512), ('conv3', 1, 512, 2048))),),
         [conv5_b1_down_w, conv5_b1_down_s, conv5_b1_conv1_w, conv5_b1_conv1_s,
          conv5_b1_conv2_w, conv5_b1_conv2_s,
          conv5_b1_conv3_w, conv5_b1_conv3_s], 8),
        ((('bneck', (('conv1', 1, 2048, 512), ('conv2', 3, 512, 512),
                     ('conv3', 1, 512, 2048))),),
         [conv5_b2_conv1_w, conv5_b2_conv1_s, conv5_b2_conv2_w,
          conv5_b2_conv2_s, conv5_b2_conv3_w, conv5_b2_conv3_s], 8),
    ]
    for blocks, ws, bn in segs:
        h = _run_segment(h, blocks, ws, bn)

    hf = jnp.transpose(h, (0, 3, 1, 2)).reshape(n, 8 * 8 * 2048)
    y = _fc_stream(hf, fc_w, fc_s)
    out = _head(y, fc1_w, fc1_s, fc2_w, fc2_s)
    return out[:, :20]
